# baseline (speedup 1.0000x reference)
import jax
import jax.numpy as jnp
from jax import lax
from jax.experimental import pallas as pl
from jax.experimental.pallas import tpu as pltpu
from jax.experimental.pallas import tpu_sc as plsc

VOCAB = 1000000
EMBED_DIM = 32
BATCH = 16384
HIST = 50

NUM_WORKERS = 32
TOTAL_ROWS = BATCH * HIST
ROWS_PER_WORKER = TOTAL_ROWS // NUM_WORKERS
CHUNK = 512
GROUP = 512
NCH = GROUP // CHUNK
SLOTS = 5
NG = ROWS_PER_WORKER // GROUP
NOUT = NG // SLOTS
CHUNKS_PER_WORKER = ROWS_PER_WORKER // CHUNK


def _glove_sc(idx_hbm, table_hbm, out_hbm, idx_v, rows_v, sem_g, sem_o):
    wid = lax.axis_index("s") * 2 + lax.axis_index("c")
    base = wid * ROWS_PER_WORKER

    pltpu.sync_copy(idx_hbm.at[wid], idx_v)

    def fire(g, slot):
        for j in range(NCH):
            pltpu.async_copy(
                table_hbm.at[idx_v.at[g * NCH + j]],
                rows_v.at[slot].at[pl.ds(j * CHUNK, CHUNK)],
                sem_g.at[slot],
            )

    def drain_gathers(slot):
        pltpu.make_async_copy(
            out_hbm.at[pl.ds(0, GROUP)], rows_v.at[slot], sem_g.at[slot]
        ).wait()

    def out_start(g, slot):
        pltpu.async_copy(
            rows_v.at[slot],
            out_hbm.at[pl.ds(base + g * GROUP, GROUP)],
            sem_o.at[slot],
        )

    def out_wait(slot):
        pltpu.make_async_copy(
            rows_v.at[slot], out_hbm.at[pl.ds(0, GROUP)], sem_o.at[slot]
        ).wait()

    for slot in range(SLOTS):
        fire(slot, slot)

    def body(i, _):
        gbase = i * SLOTS
        for slot in range(SLOTS):
            drain_gathers(slot)
            out_start(gbase + slot, slot)
        for slot in range(SLOTS):
            out_wait(slot)
            fire(gbase + slot + SLOTS, slot)
        return ()

    lax.fori_loop(0, NOUT - 1, body, (), unroll=False)

    gbase = (NOUT - 1) * SLOTS
    for slot in range(SLOTS):
        drain_gathers(slot)
        out_start(gbase + slot, slot)
    for slot in range(SLOTS):
        out_wait(slot)


@jax.jit
def kernel(idx, weights):
    idx3 = idx.reshape(NUM_WORKERS, CHUNKS_PER_WORKER, CHUNK)
    mesh = plsc.VectorSubcoreMesh(core_axis_name="c", subcore_axis_name="s")
    out = pl.kernel(
        _glove_sc,
        out_type=jax.ShapeDtypeStruct((TOTAL_ROWS, EMBED_DIM), jnp.float32),
        mesh=mesh,
        scratch_types=[
            pltpu.VMEM((CHUNKS_PER_WORKER, CHUNK), jnp.int32),
            pltpu.VMEM((SLOTS, GROUP, EMBED_DIM), jnp.float32),
            pltpu.SemaphoreType.DMA((SLOTS,)),
            pltpu.SemaphoreType.DMA((SLOTS,)),
        ],
        compiler_params=pltpu.CompilerParams(use_tc_tiling_on_sc=False),
    )(idx3, weights)
    return out.reshape(BATCH, HIST, EMBED_DIM)


# speedup vs baseline: 1.6207x; 1.6207x over previous
import jax
import jax.numpy as jnp
from jax import lax
from jax.experimental import pallas as pl
from jax.experimental.pallas import tpu as pltpu
from jax.experimental.pallas import tpu_sc as plsc

VOCAB = 1000000
EMBED_DIM = 32
BATCH = 16384
HIST = 50

NUM_WORKERS = 32
B_PER_W = BATCH // NUM_WORKERS
GB = 8
SLOTS = 4
NG = B_PER_W // GB
NOUT = NG // SLOTS


def _glove_sc(idx_hbm, table_hbm, out_hbm, idx_v, rows_v, sem_g, sem_o):
    wid = lax.axis_index("s") * 2 + lax.axis_index("c")
    b0 = wid * B_PER_W

    pltpu.sync_copy(idx_hbm.at[pl.ds(b0, B_PER_W)], idx_v)

    def fire(g, slot):
        for j in range(GB):
            pltpu.async_copy(
                table_hbm.at[idx_v.at[g * GB + j]],
                rows_v.at[slot, j],
                sem_g.at[slot],
            )

    def drain_gathers(slot):
        pltpu.make_async_copy(
            out_hbm.at[pl.ds(0, GB)], rows_v.at[slot], sem_g.at[slot]
        ).wait()

    def out_start(g, slot):
        pltpu.async_copy(
            rows_v.at[slot],
            out_hbm.at[pl.ds(b0 + g * GB, GB)],
            sem_o.at[slot],
        )

    def out_wait(slot):
        pltpu.make_async_copy(
            rows_v.at[slot], out_hbm.at[pl.ds(0, GB)], sem_o.at[slot]
        ).wait()

    for slot in range(SLOTS):
        fire(slot, slot)

    def body(i, _):
        gbase = i * SLOTS
        for slot in range(SLOTS):
            drain_gathers(slot)
            out_start(gbase + slot, slot)
        for slot in range(SLOTS):
            out_wait(slot)
            fire(gbase + slot + SLOTS, slot)
        return ()

    lax.fori_loop(0, NOUT - 1, body, (), unroll=False)

    gbase = (NOUT - 1) * SLOTS
    for slot in range(SLOTS):
        drain_gathers(slot)
        out_start(gbase + slot, slot)
    for slot in range(SLOTS):
        out_wait(slot)


@jax.jit
def kernel(idx, weights):
    mesh = plsc.VectorSubcoreMesh(core_axis_name="c", subcore_axis_name="s")
    return pl.kernel(
        _glove_sc,
        out_type=jax.ShapeDtypeStruct((BATCH, HIST, EMBED_DIM), jnp.float32),
        mesh=mesh,
        scratch_types=[
            pltpu.VMEM((B_PER_W, HIST), jnp.int32),
            pltpu.VMEM((SLOTS, GB, HIST, EMBED_DIM), jnp.float32),
            pltpu.SemaphoreType.DMA((SLOTS,)),
            pltpu.SemaphoreType.DMA((SLOTS,)),
        ],
        compiler_params=pltpu.CompilerParams(use_tc_tiling_on_sc=False),
    )(idx, weights)
